# baseline (speedup 1.0000x reference)
import jax
import jax.numpy as jnp
from jax.experimental import pallas as pl
from jax.experimental.pallas import tpu as pltpu


def _gcn_body(adj_a_ref, adj_b_ref, x_ref, w_ref, b_ref, out_ref):
    x = x_ref[...]
    w = w_ref[...]
    b = b_ref[...]
    sup_a = jnp.dot(adj_a_ref[0], x, preferred_element_type=jnp.float32)
    out_ref[0] = jnp.dot(sup_a, w, preferred_element_type=jnp.float32) + b
    sup_b = jnp.dot(adj_b_ref[0], x, preferred_element_type=jnp.float32)
    out_ref[1] = jnp.dot(sup_b, w, preferred_element_type=jnp.float32) + b


def kernel(input, adj, weight, bias):
    n_rows, f_in = input.shape
    f_out = weight.shape[1]
    n_dst = adj.shape[0]
    half = n_dst // 2
    bm = 200

    adj3 = adj.reshape(2, half, n_rows)
    out = pl.pallas_call(
        _gcn_body,
        grid=(half // bm,),
        in_specs=[
            pl.BlockSpec((1, bm, n_rows), lambda i: (0, i, 0)),
            pl.BlockSpec((1, bm, n_rows), lambda i: (1, i, 0)),
            pl.BlockSpec((n_rows, f_in), lambda i: (0, 0)),
            pl.BlockSpec((f_in, f_out), lambda i: (0, 0)),
            pl.BlockSpec((1, f_out), lambda i: (0, 0)),
        ],
        out_specs=pl.BlockSpec((2, bm, f_out), lambda i: (0, i, 0)),
        out_shape=jax.ShapeDtypeStruct((2, half, f_out), jnp.float32),
        compiler_params=pltpu.CompilerParams(
            dimension_semantics=("parallel",),
        ),
    )(adj3, adj3, input, weight, bias.reshape(1, f_out))
    return out.reshape(n_dst, f_out)


# speedup vs baseline: 1.0887x; 1.0887x over previous
import jax
import jax.numpy as jnp
from jax.experimental import pallas as pl
from jax.experimental.pallas import tpu as pltpu


def _gcn_body(adj_ref, x_ref, w_ref, b_ref, out_ref):
    support = jnp.dot(adj_ref[...], x_ref[...],
                      preferred_element_type=jnp.float32)
    out_ref[...] = jnp.dot(support, w_ref[...],
                           preferred_element_type=jnp.float32) + b_ref[...]


def kernel(input, adj, weight, bias):
    n_rows, f_in = input.shape
    f_out = weight.shape[1]
    n_dst = adj.shape[0]
    bm = 400

    out = pl.pallas_call(
        _gcn_body,
        grid=(n_dst // bm,),
        in_specs=[
            pl.BlockSpec((bm, n_rows), lambda i: (i, 0)),
            pl.BlockSpec((n_rows, f_in), lambda i: (0, 0)),
            pl.BlockSpec((f_in, f_out), lambda i: (0, 0)),
            pl.BlockSpec((1, f_out), lambda i: (0, 0)),
        ],
        out_specs=pl.BlockSpec((bm, f_out), lambda i: (i, 0)),
        out_shape=jax.ShapeDtypeStruct((n_dst, f_out), jnp.float32),
        compiler_params=pltpu.CompilerParams(
            dimension_semantics=("parallel",),
            vmem_limit_bytes=64 * 1024 * 1024,
        ),
    )(adj, input, weight, bias.reshape(1, f_out))
    return out
